# baseline (speedup 1.0000x reference)
import jax
import jax.numpy as jnp
from jax.experimental import pallas as pl
from jax.experimental.pallas import tpu as pltpu

P = 256
D = 16
ED = 16
BB = 4


def _dot(a, b, dims=((1,), (0,))):
    return jax.lax.dot_general(a, b, (dims, ((), ())),
                               preferred_element_type=jnp.float32)


def _gin_kernel(nodes_ref, We1_ref, We1T_ref, We2T_ref, wd_ref,
                be_ref, bec_ref, Wn1_ref, Wn2_ref, bn_ref, out_ref, agg_scr):
    rows = jax.lax.broadcasted_iota(jnp.int32, (P, P), 0)
    cols = jax.lax.broadcasted_iota(jnp.int32, (P, P), 1)
    offdiag = (rows != cols).astype(jnp.float32)

    for i in range(BB):
        nodes = nodes_ref[i]
        nT = jnp.transpose(nodes)

        g = _dot(nodes, nT)
        sq_row = jnp.sum(nT * nT, axis=0, keepdims=True)
        sq_col = jnp.sum(nodes * nodes, axis=1, keepdims=True)
        dist = jnp.sqrt(jnp.maximum(sq_col + sq_row - 2.0 * g, 0.0)) * offdiag

        a2 = _dot(nodes, We1_ref[...]) + be_ref[...]
        a2T = _dot(We1T_ref[...], nT) + bec_ref[...]
        cT = _dot(We2T_ref[...], nT)

        for k in range(ED):
            m = dist * wd_ref[0, k] + a2[:, k:k + 1] + cT[k:k + 1, :]
            m = jnp.maximum(m, 0.0)
            agg_scr[k:k + 1, :] = jnp.sum(m, axis=0, keepdims=True)

        aggT = agg_scr[...] - jnp.maximum(a2T + cT, 0.0)
        agg = jnp.transpose(aggT)

        out_ref[i] = (_dot(agg, Wn1_ref[...])
                      + _dot(nodes, Wn2_ref[...])
                      + bn_ref[...])


def kernel(t, h, W_e, b_e, W_n, b_n):
    del t
    B = h.shape[0]
    nodes = h.reshape(B, P, D)

    We1 = W_e[:D]
    We1T = We1.T
    We2T = W_e[D:2 * D].T
    wd = W_e[2 * D:2 * D + 1]
    be = b_e.reshape(1, ED)
    bec = b_e.reshape(ED, 1)
    Wn1 = W_n[:ED]
    Wn2 = W_n[ED:]
    bn = b_n.reshape(1, D)

    full = lambda shape: pl.BlockSpec(shape, lambda b: (0,) * len(shape))
    out = pl.pallas_call(
        _gin_kernel,
        grid=(B // BB,),
        in_specs=[
            pl.BlockSpec((BB, P, D), lambda b: (b, 0, 0)),
            full((D, ED)), full((ED, D)), full((ED, D)), full((1, ED)),
            full((1, ED)), full((ED, 1)), full((ED, D)), full((D, D)),
            full((1, D)),
        ],
        out_specs=pl.BlockSpec((BB, P, D), lambda b: (b, 0, 0)),
        out_shape=jax.ShapeDtypeStruct((B, P, D), jnp.float32),
        scratch_shapes=[pltpu.VMEM((ED, P), jnp.float32)],
        compiler_params=pltpu.CompilerParams(
            dimension_semantics=("parallel",)),
    )(nodes, We1, We1T, We2T, wd, be, bec, Wn1, Wn2, bn)

    return out.reshape(B, P * D)
